# baseline (speedup 1.0000x reference)
import functools

import jax
import jax.numpy as jnp
from jax import lax
from jax.experimental import pallas as pl
from jax.experimental.pallas import tpu as pltpu
from jax.experimental.pallas import tpu_sc as plsc

_E = 3200000
_EI = 2 * _E
_NW = 32
_WI = _EI // _NW
_WV = _E // _NW
_C = 20000
_LANES = 16

_mesh = plsc.VectorSubcoreMesh(core_axis_name="c", subcore_axis_name="s")


def _iview(a):
    return a.reshape(_EI // 256, 128, 2).swapaxes(1, 2).reshape(_EI)


@functools.partial(
    pl.kernel,
    out_type=[
        jax.ShapeDtypeStruct((2 * _EI,), jnp.int32),
        jax.ShapeDtypeStruct((2 * _E,), jnp.int32),
    ],
    mesh=_mesh,
    scratch_types=[
        pltpu.VMEM((_C,), jnp.int32),
        pltpu.VMEM((_C,), jnp.int32),
        pltpu.VMEM((_C,), jnp.int32),
        pltpu.VMEM((_LANES,), jnp.int32),
        pltpu.SemaphoreType.DMA((3,)),
        pltpu.SemaphoreType.DMA,
    ],
)
def _sc_concat(a1i_h, a2i_h, a1v_h, a2v_h, m_h, oi_h, ov_h,
               buf0, buf1, buf2, mbuf, sem_out, sem_in):
    wid = lax.axis_index("s") * 2 + lax.axis_index("c")
    ib = wid * _WI
    vb = wid * _WV

    pltpu.sync_copy(m_h, mbuf)
    m = mbuf[...]

    bufs = (buf0, buf1, buf2)

    streams = [
        (a2i_h, ib, oi_h, _EI + ib, _WI, True),
        (a1i_h, ib, oi_h, ib, _WI, False),
        (a1v_h, vb, ov_h, vb, _WV, False),
        (a2v_h, vb, ov_h, _E + vb, _WV, False),
    ]
    chunks = []
    for src, sbase, dst, dbase, n, add in streams:
        for c in range(n // _C):
            chunks.append((src, sbase + c * _C, dst, dbase + c * _C, add))
    ntot = len(chunks)

    def in_copy(k):
        src, soff, _, _, _ = chunks[k]
        return pltpu.make_async_copy(src.at[pl.ds(soff, _C)], bufs[k % 3],
                                     sem_in)

    def out_copy(k):
        _, _, dst, doff, _ = chunks[k]
        return pltpu.make_async_copy(bufs[k % 3], dst.at[pl.ds(doff, _C)],
                                     sem_out.at[k % 3])

    in_copy(0).start()

    for k in range(ntot):
        b = bufs[k % 3]
        in_copy(k).wait()
        if k >= 2:
            out_copy(k - 2).wait()
        if k + 1 < ntot:
            in_copy(k + 1).start()

        if chunks[k][4]:
            def add_body(j, _):
                sl = pl.ds(j * _LANES, _LANES)
                b[sl] = b[sl] + m
                return _

            lax.fori_loop(0, _C // _LANES, add_body, 0)
        out_copy(k).start()

    out_copy(ntot - 2).wait()
    out_copy(ntot - 1).wait()


def kernel(a1_indices, a1_values, a2_indices, a2_values, M):
    a1i = _iview(a1_indices)
    a2i = _iview(a2_indices)
    a1v = lax.bitcast_convert_type(a1_values, jnp.int32)
    a2v = lax.bitcast_convert_type(a2_values, jnp.int32)
    mvec = jnp.full((_LANES,), M, jnp.int32)

    oi, ov = _sc_concat(a1i, a2i, a1v, a2v, mvec)

    new_inds = (oi.reshape(2 * _EI // 256, 2, 128)
                  .swapaxes(1, 2)
                  .reshape(_EI, 2))
    new_vals = lax.bitcast_convert_type(ov, jnp.float32)
    return new_inds, new_vals


# speedup vs baseline: 1.5280x; 1.5280x over previous
import functools

import jax
import jax.numpy as jnp
from jax import lax
from jax.experimental import pallas as pl
from jax.experimental.pallas import tpu as pltpu
from jax.experimental.pallas import tpu_sc as plsc

_E = 3200000
_EI = 2 * _E
_RI = _E // 64
_G = 5
_BI = _RI // _G

_NW = 32
_WV = _E // _NW
_C = 25000
_LANES = 16

_mesh = plsc.VectorSubcoreMesh(core_axis_name="c", subcore_axis_name="s")


def _iview(a):
    return a.reshape(_RI // 2, 128, 2).swapaxes(1, 2).reshape(_RI, 128)


def _tc_body(m_ref, a1i, a2i, oi):
    oi[0] = a1i[...]
    oi[1] = a2i[...] + m_ref[0]


@functools.partial(
    pl.kernel,
    out_type=jax.ShapeDtypeStruct((2 * _E,), jnp.int32),
    mesh=_mesh,
    scratch_types=[
        pltpu.VMEM((_C,), jnp.int32),
        pltpu.VMEM((_C,), jnp.int32),
        pltpu.VMEM((_C,), jnp.int32),
        pltpu.SemaphoreType.DMA((3,)),
        pltpu.SemaphoreType.DMA,
    ],
)
def _sc_vals(a1v_h, a2v_h, ov_h, buf0, buf1, buf2, sem_out, sem_in):
    wid = lax.axis_index("s") * 2 + lax.axis_index("c")
    vb = wid * _WV

    bufs = (buf0, buf1, buf2)
    chunks = []
    for src, sbase, dbase in ((a1v_h, vb, vb), (a2v_h, vb, _E + vb)):
        for c in range(_WV // _C):
            chunks.append((src, sbase + c * _C, dbase + c * _C))
    ntot = len(chunks)

    def in_copy(k):
        src, soff, _ = chunks[k]
        return pltpu.make_async_copy(src.at[pl.ds(soff, _C)], bufs[k % 3],
                                     sem_in)

    def out_copy(k):
        _, _, doff = chunks[k]
        return pltpu.make_async_copy(bufs[k % 3], ov_h.at[pl.ds(doff, _C)],
                                     sem_out.at[k % 3])

    in_copy(0).start()
    for k in range(ntot):
        in_copy(k).wait()
        if k >= 2:
            out_copy(k - 2).wait()
        if k + 1 < ntot:
            in_copy(k + 1).start()
        out_copy(k).start()
    out_copy(ntot - 2).wait()
    out_copy(ntot - 1).wait()


def kernel(a1_indices, a1_values, a2_indices, a2_values, M):
    idt = a1_indices.dtype
    a1i = _iview(a1_indices)
    a2i = _iview(a2_indices)
    m = jnp.asarray(M, idt).reshape(1)

    oi = pl.pallas_call(
        _tc_body,
        grid=(_G,),
        in_specs=[
            pl.BlockSpec(memory_space=pltpu.SMEM),
            pl.BlockSpec((_BI, 128), lambda i: (i, 0)),
            pl.BlockSpec((_BI, 128), lambda i: (i, 0)),
        ],
        out_specs=pl.BlockSpec((2, _BI, 128), lambda i: (0, i, 0)),
        out_shape=jax.ShapeDtypeStruct((2, _RI, 128), idt),
    )(m, a1i, a2i)

    a1v = lax.bitcast_convert_type(a1_values, jnp.int32)
    a2v = lax.bitcast_convert_type(a2_values, jnp.int32)
    ov = _sc_vals(a1v, a2v)

    new_inds = (oi.reshape(2 * _RI // 2, 2, 128)
                  .swapaxes(1, 2)
                  .reshape(2 * _E, 2))
    new_vals = lax.bitcast_convert_type(ov, jnp.float32)
    return new_inds, new_vals


# speedup vs baseline: 2.9512x; 1.9315x over previous
import jax
import jax.numpy as jnp
from jax.experimental import pallas as pl
from jax.experimental.pallas import tpu as pltpu

_E = 3200000
_RI = _E // 64
_RV = _E // 128
_G = 25
_BI = _RI // _G
_BV = _RV // _G
_H = _BI // 2


def _iview(a):
    return a.reshape(_RI // 2, 128, 2).swapaxes(1, 2).reshape(_RI, 128)


def _out_copies(i, s, obuf, vbuf, oi, ov, sem):
    return [
        pltpu.make_async_copy(obuf.at[s, 0, pl.ds(0, _H)],
                              oi.at[0].at[pl.ds(i * _BI, _H)], sem.at[s, 0]),
        pltpu.make_async_copy(obuf.at[s, 0, pl.ds(_H, _H)],
                              oi.at[0].at[pl.ds(i * _BI + _H, _H)],
                              sem.at[s, 1]),
        pltpu.make_async_copy(obuf.at[s, 1, pl.ds(0, _H)],
                              oi.at[1].at[pl.ds(i * _BI, _H)], sem.at[s, 2]),
        pltpu.make_async_copy(obuf.at[s, 1, pl.ds(_H, _H)],
                              oi.at[1].at[pl.ds(i * _BI + _H, _H)],
                              sem.at[s, 3]),
        pltpu.make_async_copy(vbuf.at[s, 0],
                              ov.at[0].at[pl.ds(i * _BV, _BV)], sem.at[s, 4]),
        pltpu.make_async_copy(vbuf.at[s, 1],
                              ov.at[1].at[pl.ds(i * _BV, _BV)], sem.at[s, 5]),
    ]


def _body(m_ref, a1i, a2i, a1v, a2v, oi, ov, obuf, vbuf, sem):
    i = pl.program_id(0)
    s = jax.lax.rem(i, 2)

    @pl.when(i >= 2)
    def _wait_prev():
        for c in _out_copies(i - 2, s, obuf, vbuf, oi, ov, sem):
            c.wait()

    obuf[s, 0] = a1i[...]
    obuf[s, 1] = a2i[...] + m_ref[0]
    vbuf[s, 0] = a1v[...]
    vbuf[s, 1] = a2v[...]
    for c in _out_copies(i, s, obuf, vbuf, oi, ov, sem):
        c.start()

    @pl.when(i == _G - 1)
    def _drain():
        for c in _out_copies(i - 1, 1 - s, obuf, vbuf, oi, ov, sem):
            c.wait()
        for c in _out_copies(i, s, obuf, vbuf, oi, ov, sem):
            c.wait()


def kernel(a1_indices, a1_values, a2_indices, a2_values, M):
    idt = a1_indices.dtype
    a1i = _iview(a1_indices)
    a2i = _iview(a2_indices)
    a1v = a1_values.reshape(_RV, 128)
    a2v = a2_values.reshape(_RV, 128)
    m = jnp.asarray(M, idt).reshape(1)

    oi, ov = pl.pallas_call(
        _body,
        grid=(_G,),
        in_specs=[
            pl.BlockSpec(memory_space=pltpu.SMEM),
            pl.BlockSpec((_BI, 128), lambda i: (i, 0)),
            pl.BlockSpec((_BI, 128), lambda i: (i, 0)),
            pl.BlockSpec((_BV, 128), lambda i: (i, 0)),
            pl.BlockSpec((_BV, 128), lambda i: (i, 0)),
        ],
        out_specs=[
            pl.BlockSpec(memory_space=pltpu.MemorySpace.HBM),
            pl.BlockSpec(memory_space=pltpu.MemorySpace.HBM),
        ],
        out_shape=[
            jax.ShapeDtypeStruct((2, _RI, 128), idt),
            jax.ShapeDtypeStruct((2, _RV, 128), a1_values.dtype),
        ],
        scratch_shapes=[
            pltpu.VMEM((2, 2, _BI, 128), idt),
            pltpu.VMEM((2, 2, _BV, 128), jnp.float32),
            pltpu.SemaphoreType.DMA((2, 6)),
        ],
    )(m, a1i, a2i, a1v, a2v)

    new_inds = (oi.reshape(2 * _RI // 2, 2, 128)
                  .swapaxes(1, 2)
                  .reshape(2 * _E, 2))
    new_vals = ov.reshape(2 * _E)
    return new_inds, new_vals


# speedup vs baseline: 3.1440x; 1.0653x over previous
import jax
import jax.numpy as jnp
from jax.experimental import pallas as pl
from jax.experimental.pallas import tpu as pltpu

_E = 3200000
_RI = _E // 64
_RV = _E // 128
_G = 5
_BI = _RI // _G
_BV = _RV // _G


def _iview(a):
    return a.reshape(_RI // 2, 128, 2).swapaxes(1, 2).reshape(_RI, 128)


def _body(m_ref, a1i, a2i, a1v, a2v, oi, ov):
    oi[0] = a1i[...]
    oi[1] = a2i[...] + m_ref[0]
    ov[0] = a1v[...]
    ov[1] = a2v[...]


def kernel(a1_indices, a1_values, a2_indices, a2_values, M):
    idt = a1_indices.dtype
    a1i = _iview(a1_indices)
    a2i = _iview(a2_indices)
    a1v = a1_values.reshape(_RV, 128)
    a2v = a2_values.reshape(_RV, 128)
    m = jnp.asarray(M, idt).reshape(1)

    oi, ov = pl.pallas_call(
        _body,
        grid=(_G,),
        in_specs=[
            pl.BlockSpec(memory_space=pltpu.SMEM),
            pl.BlockSpec((_BI, 128), lambda i: (i, 0)),
            pl.BlockSpec((_BI, 128), lambda i: (i, 0)),
            pl.BlockSpec((_BV, 128), lambda i: (i, 0)),
            pl.BlockSpec((_BV, 128), lambda i: (i, 0)),
        ],
        out_specs=[
            pl.BlockSpec((2, _BI, 128), lambda i: (0, i, 0)),
            pl.BlockSpec((2, _BV, 128), lambda i: (0, i, 0)),
        ],
        out_shape=[
            jax.ShapeDtypeStruct((2, _RI, 128), idt),
            jax.ShapeDtypeStruct((2, _RV, 128), a1_values.dtype),
        ],
    )(m, a1i, a2i, a1v, a2v)

    new_inds = (oi.reshape(2 * _RI // 2, 2, 128)
                  .swapaxes(1, 2)
                  .reshape(2 * _E, 2))
    new_vals = ov.reshape(2 * _E)
    return new_inds, new_vals
